# baseline (speedup 1.0000x reference)
import functools

import numpy as np
import jax
import jax.numpy as jnp
from jax import lax
from jax.experimental import pallas as pl
from jax.experimental.pallas import tpu as pltpu
from jax.experimental.pallas import tpu_sc as plsc

N = 10000
E = 320000
IN_DIM = 128
H1 = 8
C1 = 8
NCLASS = 10

NC = 2
NS = 16
NW = NC * NS
EPW = E // NW
K = 80
NCHUNK = EPW // K
RPT = N // NS

_f32 = jnp.float32
_i32 = jnp.int32


def _edge_tables(row_w, H, C):
    hc = H * C
    as_w0 = row_w - 16 if hc >= 16 else 0
    lane0 = hc - as_w0
    q = np.arange(K * row_w, dtype=np.int32)
    e = q // row_w
    col = q % row_w
    head = np.where(col < hc, col // C,
                    np.where(col < hc + H, col - hc, 0))
    t_all = e * 16 + lane0 + head
    return as_w0, lane0, t_all.astype(np.int32)


def _make_edge_pass(row_w, H, C, stage_tables=False):
    W16 = row_w // 16
    AS_W0, _, _ = _edge_tables(row_w, H, C)
    assert NCHUNK % 2 == 1
    mesh = plsc.VectorSubcoreMesh(core_axis_name="c", subcore_axis_name="s")
    shared_scratch = (
        [pltpu.VMEM_SHARED((N, row_w), _f32),
         pltpu.VMEM_SHARED((N, 16), _f32)]
        if stage_tables else [])

    @functools.partial(
        pl.kernel,
        mesh=mesh,
        out_type=jax.ShapeDtypeStruct((NC, N, row_w), _f32),
        compiler_params=pltpu.CompilerParams(
            use_tc_tiling_on_sc=False, needs_layout_passes=False),
        scratch_types=shared_scratch + [
            pltpu.VMEM_SHARED((N, row_w), _f32),
            pltpu.VMEM((EPW,), _i32),
            pltpu.VMEM((EPW,), _i32),
            pltpu.VMEM((K, row_w), _f32),
            pltpu.VMEM((K, row_w), _f32),
            pltpu.VMEM((K, row_w), _f32),
            pltpu.VMEM((K, row_w), _f32),
            pltpu.VMEM((K, 16), _f32),
            pltpu.VMEM((K, 16), _f32),
            pltpu.VMEM((K, 16), _f32),
            pltpu.VMEM((K, 16), _f32),
            pltpu.VMEM((K * 16,), _f32),
            pltpu.VMEM((K, row_w), _f32),
            pltpu.VMEM((K, row_w), _f32),
            pltpu.VMEM((16,), _f32),
            pltpu.VMEM((K * row_w,), _i32),
            pltpu.SemaphoreType.DMA,
            pltpu.SemaphoreType.DMA,
            pltpu.SemaphoreType.DMA,
            pltpu.SemaphoreType.DMA,
            pltpu.SemaphoreType.DMA,
            pltpu.SemaphoreType.DMA,
        ],
    )
    def edge_pass(src_hbm, dst_hbm, tab_hbm, adt_hbm, bound_hbm,
                  tall_hbm,
                  out_hbm,
                  *scratch):
        if stage_tables:
            (tab_src, adt_src, acc_sh, sidx_all, didx_all,
             srows_0, srows_1, srows_2, srows_3,
             adrows_0, adrows_1, adrows_2, adrows_3,
             exbuf, staging_a, staging_b, boundv, tall,
             sem_0, sem_1, sem_2, sem_3, sem_sa, sem_sb) = scratch
        else:
            (acc_sh, sidx_all, didx_all,
             srows_0, srows_1, srows_2, srows_3,
             adrows_0, adrows_1, adrows_2, adrows_3,
             exbuf, staging_a, staging_b, boundv, tall,
             sem_0, sem_1, sem_2, sem_3, sem_sa, sem_sb) = scratch
            tab_src, adt_src = tab_hbm, adt_hbm
        cid = lax.axis_index("c")
        sid = lax.axis_index("s")
        wid = sid * NC + cid
        row0 = sid * RPT
        ebase = wid * EPW

        pltpu.sync_copy(bound_hbm, boundv)
        pltpu.sync_copy(tall_hbm, tall)
        pltpu.sync_copy(src_hbm.at[pl.ds(ebase, EPW)], sidx_all)
        pltpu.sync_copy(dst_hbm.at[pl.ds(ebase, EPW)], didx_all)
        if stage_tables:
            pltpu.sync_copy(tab_hbm.at[pl.ds(row0, RPT)],
                            tab_src.at[pl.ds(row0, RPT)])
            pltpu.sync_copy(adt_hbm.at[pl.ds(row0, RPT)],
                            adt_src.at[pl.ds(row0, RPT)])

        zero16 = jnp.zeros((16,), _f32)

        def zrow(r, carry):
            for j in range(W16):
                staging_a[r, pl.ds(j * 16, 16)] = zero16
            return carry

        lax.fori_loop(0, K, zrow, 0)
        nfull = RPT // K
        rem = RPT - nfull * K
        for r in range(nfull):
            pltpu.sync_copy(staging_a, acc_sh.at[pl.ds(row0 + r * K, K)])
        if rem:
            pltpu.sync_copy(staging_a.at[pl.ds(0, rem)],
                            acc_sh.at[pl.ds(row0 + nfull * K, rem)])
        plsc.subcore_barrier()

        bvec = boundv[...]
        lane = lax.broadcasted_iota(_i32, (16,), 0)

        def fire(c, srows, adrows, sem):
            sl = pl.ds(c * K, K)
            pltpu.async_copy(tab_src.at[sidx_all.at[sl]], srows, sem)
            pltpu.async_copy(adt_src.at[didx_all.at[sl]], adrows, sem)

        def drain(srows, adrows, sem):
            pltpu.make_async_copy(tab_hbm.at[pl.ds(0, K)], srows, sem).wait()
            pltpu.make_async_copy(adt_hbm.at[pl.ds(0, K)], adrows, sem).wait()

        def scat(c, staging, sem_s):
            return pltpu.make_async_copy(
                staging, acc_sh.at[didx_all.at[pl.ds(c * K, K)]], sem_s)

        def compute_scatter(c, srows, adrows, staging, sem_s, guard):
            @pl.when(guard)
            def _():
                scat(c, staging, sem_s).wait()

            @plsc.parallel_loop(0, K, unroll=4)
            def exphase(e):
                asv = srows[e, pl.ds(AS_W0, 16)]
                adv = adrows[e, pl.ds(0, 16)]
                ev = asv + adv
                ev = jnp.where(ev > 0, ev, 0.2 * ev)
                exbuf[pl.ds(e * 16, 16)] = jnp.exp(ev - bvec)

            @plsc.parallel_loop(0, K, unroll=2)
            def msg(e):
                for j in range(W16):
                    lo = j * 16
                    colv = lane + lo
                    idxv = tall[pl.ds(e * row_w + lo, 16)]
                    exg = plsc.load_gather(exbuf, [idxv])
                    if lo + 16 <= H * C:
                        val = srows[e, pl.ds(lo, 16)] * exg
                    elif lo >= H * C:
                        val = jnp.where(colv < H * C + H, exg, 0.0)
                    else:
                        hv = srows[e, pl.ds(lo, 16)]
                        val = jnp.where(colv < H * C, hv * exg,
                                        jnp.where(colv < H * C + H, exg, 0.0))
                    staging[e, pl.ds(lo, 16)] = val

            pltpu.async_copy(staging,
                             acc_sh.at[didx_all.at[pl.ds(c * K, K)]],
                             sem_s, add=True)

        bufs = [(srows_0, adrows_0, sem_0), (srows_1, adrows_1, sem_1),
                (srows_2, adrows_2, sem_2), (srows_3, adrows_3, sem_3)]
        stgs = [(staging_a, sem_sa), (staging_b, sem_sb)]
        DEPTH = 4
        for b in range(DEPTH):
            fire(b, *bufs[b])

        def quad(i, carry):
            for b in range(DEPTH):
                c = DEPTH * i + b
                drain(*bufs[b])
                stg, ssem = stgs[b % 2]
                guard = (i > 0) if b < 2 else jnp.bool_(True)
                compute_scatter(c, bufs[b][0], bufs[b][1], stg, ssem, guard)

                @pl.when(c + DEPTH < NCHUNK)
                def _(sb=bufs[b], cc=c):
                    fire(cc + DEPTH, *sb)
            return carry

        nquad = NCHUNK // DEPTH
        lax.fori_loop(0, nquad, quad, 0)
        for b in range(NCHUNK - DEPTH * nquad):
            c = DEPTH * nquad + b
            drain(*bufs[b])
            stg, ssem = stgs[b % 2]
            compute_scatter(c, bufs[b][0], bufs[b][1], stg, ssem,
                            jnp.bool_(True))
        scat(0, staging_a, sem_sa).wait()
        scat(0, staging_b, sem_sb).wait()

        plsc.subcore_barrier()
        pltpu.sync_copy(acc_sh.at[pl.ds(row0, RPT)],
                        out_hbm.at[cid, pl.ds(row0, RPT)])

    return edge_pass


_edge_pass1 = _make_edge_pass(80, H1, C1, stage_tables=False)
_edge_pass2 = _make_edge_pass(16, 1, NCLASS, stage_tables=True)
_, _, _TALL1 = _edge_tables(80, H1, C1)
_, _LANE0_2, _TALL2 = _edge_tables(16, 1, NCLASS)


BK = 2000
NBLK = N // BK


def _tcA_body(z_ref, w_ref, as_ref, ad_ref, tab_ref, adt_ref, ms_ref, md_ref):
    i = pl.program_id(0)
    h = jnp.dot(z_ref[...], w_ref[...], preferred_element_type=_f32)
    rows = lax.broadcasted_iota(_i32, (H1 * C1, H1), 0)
    cols = lax.broadcasted_iota(_i32, (H1 * C1, H1), 1)
    G = jnp.where(rows // C1 == cols, 1.0, 0.0).astype(_f32)
    asn = jnp.dot(h * as_ref[...], G, preferred_element_type=_f32)
    adn = jnp.dot(h * ad_ref[...], G, preferred_element_type=_f32)
    tab_ref[...] = jnp.concatenate(
        [h, asn, jnp.zeros((BK, 8), _f32)], axis=1)
    adt_ref[...] = jnp.concatenate([adn, jnp.zeros((BK, 8), _f32)], axis=1)
    mva = jnp.max(asn, axis=0, keepdims=True)
    mvd = jnp.max(adn, axis=0, keepdims=True)

    @pl.when(i == 0)
    def _():
        ms_ref[...] = mva
        md_ref[...] = mvd

    @pl.when(i > 0)
    def _():
        ms_ref[...] = jnp.maximum(ms_ref[...], mva)
        md_ref[...] = jnp.maximum(md_ref[...], mvd)


def _tcA(z, w1, a_s_flat, a_d_flat):
    return pl.pallas_call(
        _tcA_body,
        grid=(NBLK,),
        in_specs=[
            pl.BlockSpec((BK, IN_DIM), lambda i: (i, 0)),
            pl.BlockSpec((IN_DIM, H1 * C1), lambda i: (0, 0)),
            pl.BlockSpec((1, H1 * C1), lambda i: (0, 0)),
            pl.BlockSpec((1, H1 * C1), lambda i: (0, 0)),
        ],
        out_specs=[
            pl.BlockSpec((BK, 80), lambda i: (i, 0)),
            pl.BlockSpec((BK, 16), lambda i: (i, 0)),
            pl.BlockSpec((1, H1), lambda i: (0, 0)),
            pl.BlockSpec((1, H1), lambda i: (0, 0)),
        ],
        out_shape=[
            jax.ShapeDtypeStruct((N, 80), _f32),
            jax.ShapeDtypeStruct((N, 16), _f32),
            jax.ShapeDtypeStruct((1, H1), _f32),
            jax.ShapeDtypeStruct((1, H1), _f32),
        ],
    )(z, w1, a_s_flat, a_d_flat)


def _tcB_body(acc_ref, b1_ref, w2_ref, as2_ref, ad2_ref,
              tab2_ref, ad2t_ref, ms_ref, md_ref):
    i = pl.program_id(0)
    s = acc_ref[0] + acc_ref[1]
    num = s[:, :H1 * C1]
    den = s[:, H1 * C1:H1 * C1 + H1]
    rows = lax.broadcasted_iota(_i32, (H1, H1 * C1), 0)
    cols = lax.broadcasted_iota(_i32, (H1, H1 * C1), 1)
    P = jnp.where(rows == cols // C1, 1.0, 0.0).astype(_f32)
    denb = jnp.dot(den, P, preferred_element_type=_f32)
    x = num / (denb + 1e-16) + b1_ref[...]
    x = jnp.where(x > 0, x, jnp.exp(x) - 1.0)
    h2 = jnp.dot(x, w2_ref[...], preferred_element_type=_f32)
    as2 = jnp.sum(h2 * as2_ref[...], axis=1, keepdims=True)
    ad2 = jnp.sum(h2 * ad2_ref[...], axis=1, keepdims=True)
    tab2_ref[...] = jnp.concatenate(
        [h2, as2, jnp.zeros((BK, 5), _f32)], axis=1)
    ad2t_ref[...] = jnp.concatenate(
        [jnp.zeros((BK, 10), _f32), ad2, jnp.zeros((BK, 5), _f32)], axis=1)
    mva = jnp.max(as2).reshape(1, 1)
    mvd = jnp.max(ad2).reshape(1, 1)

    @pl.when(i == 0)
    def _():
        ms_ref[...] = mva
        md_ref[...] = mvd

    @pl.when(i > 0)
    def _():
        ms_ref[...] = jnp.maximum(ms_ref[...], mva)
        md_ref[...] = jnp.maximum(md_ref[...], mvd)


def _tcB(acc1, b1_row, w2, a_src2, a_dst2):
    return pl.pallas_call(
        _tcB_body,
        grid=(NBLK,),
        in_specs=[
            pl.BlockSpec((NC, BK, 80), lambda i: (0, i, 0)),
            pl.BlockSpec((1, H1 * C1), lambda i: (0, 0)),
            pl.BlockSpec((H1 * C1, NCLASS), lambda i: (0, 0)),
            pl.BlockSpec((1, NCLASS), lambda i: (0, 0)),
            pl.BlockSpec((1, NCLASS), lambda i: (0, 0)),
        ],
        out_specs=[
            pl.BlockSpec((BK, 16), lambda i: (i, 0)),
            pl.BlockSpec((BK, 16), lambda i: (i, 0)),
            pl.BlockSpec((1, 1), lambda i: (0, 0)),
            pl.BlockSpec((1, 1), lambda i: (0, 0)),
        ],
        out_shape=[
            jax.ShapeDtypeStruct((N, 16), _f32),
            jax.ShapeDtypeStruct((N, 16), _f32),
            jax.ShapeDtypeStruct((1, 1), _f32),
            jax.ShapeDtypeStruct((1, 1), _f32),
        ],
    )(acc1, b1_row, w2, a_src2, a_dst2)


def _tcC_body(acc_ref, b2_ref, out_ref):
    s = acc_ref[0] + acc_ref[1]
    lanes = lax.broadcasted_iota(_i32, (BK, 16), 1)
    den = jnp.sum(jnp.where(lanes == NCLASS, s, 0.0), axis=1, keepdims=True)
    o = s / (den + 1e-16) + b2_ref[...]
    om = jnp.where(lanes < NCLASS, o, -1e30)
    m = jnp.max(om, axis=1, keepdims=True)
    ssum = jnp.sum(jnp.where(lanes < NCLASS, jnp.exp(o - m), 0.0),
                   axis=1, keepdims=True)
    out_ref[...] = o - (m + jnp.log(ssum))


def _tcC(acc2, b2_row):
    return pl.pallas_call(
        _tcC_body,
        grid=(NBLK,),
        in_specs=[
            pl.BlockSpec((NC, BK, 16), lambda i: (0, i, 0)),
            pl.BlockSpec((1, 16), lambda i: (0, 0)),
        ],
        out_specs=pl.BlockSpec((BK, 16), lambda i: (i, 0)),
        out_shape=jax.ShapeDtypeStruct((N, 16), _f32),
    )(acc2, b2_row)


def kernel(z, edge_index, W1, a_src1, a_dst1, b1, W2, a_src2, a_dst2, b2):
    src = edge_index[0]
    dst = edge_index[1]

    tab1, ad1t, ms1, md1 = _tcA(z, W1, a_src1.reshape(1, H1 * C1),
                                a_dst1.reshape(1, H1 * C1))
    m1 = ms1 + md1
    bound1 = jnp.where(m1 > 0, m1, 0.2 * m1).reshape(H1)
    b1_16 = jnp.tile(bound1, 2)
    acc1 = _edge_pass1(src, dst, tab1, ad1t, b1_16, _TALL1)

    tab2, ad2t, ms2, md2 = _tcB(acc1, b1.reshape(1, H1 * C1),
                                W2, a_src2, a_dst2)
    m2 = ms2 + md2
    bound2 = jnp.where(m2 > 0, m2, 0.2 * m2)
    b2_16 = jnp.broadcast_to(bound2.reshape(1), (16,))
    acc2 = _edge_pass2(src, dst, tab2, ad2t, b2_16, _TALL2)

    b2_row = jnp.concatenate([b2, jnp.zeros((6,), _f32)]).reshape(1, 16)
    outc = _tcC(acc2, b2_row)
    return outc[:, :NCLASS]


# speedup vs baseline: 1.0287x; 1.0287x over previous
import functools

import numpy as np
import jax
import jax.numpy as jnp
from jax import lax
from jax.experimental import pallas as pl
from jax.experimental.pallas import tpu as pltpu
from jax.experimental.pallas import tpu_sc as plsc

N = 10000
E = 320000
IN_DIM = 128
H1 = 8
C1 = 8
NCLASS = 10

NC = 2
NS = 16
NW = NC * NS
EPW = E // NW
K = 80
NCHUNK = EPW // K
RPT = N // NS

_f32 = jnp.float32
_i32 = jnp.int32


def _edge_tables(row_w, H, C):
    hc = H * C
    as_w0 = row_w - 16 if hc >= 16 else 0
    lane0 = hc - as_w0
    q = np.arange(K * row_w, dtype=np.int32)
    e = q // row_w
    col = q % row_w
    head = np.where(col < hc, col // C,
                    np.where(col < hc + H, col - hc, 0))
    t_all = e * 16 + lane0 + head
    return as_w0, lane0, t_all.astype(np.int32)


def _make_edge_pass(row_w, H, C, stage_tables=False):
    W16 = row_w // 16
    AS_W0, _, _ = _edge_tables(row_w, H, C)
    assert NCHUNK % 2 == 1
    mesh = plsc.VectorSubcoreMesh(core_axis_name="c", subcore_axis_name="s")
    shared_scratch = (
        [pltpu.VMEM_SHARED((N, row_w), _f32),
         pltpu.VMEM_SHARED((N, 16), _f32)]
        if stage_tables else [])

    @functools.partial(
        pl.kernel,
        mesh=mesh,
        out_type=jax.ShapeDtypeStruct((NC, N, row_w), _f32),
        compiler_params=pltpu.CompilerParams(
            use_tc_tiling_on_sc=False, needs_layout_passes=False),
        scratch_types=shared_scratch + [
            pltpu.VMEM_SHARED((N, row_w), _f32),
            pltpu.VMEM((EPW,), _i32),
            pltpu.VMEM((EPW,), _i32),
            pltpu.VMEM((K, row_w), _f32),
            pltpu.VMEM((K, row_w), _f32),
            pltpu.VMEM((K, row_w), _f32),
            pltpu.VMEM((K, row_w), _f32),
            pltpu.VMEM((K, 16), _f32),
            pltpu.VMEM((K, 16), _f32),
            pltpu.VMEM((K, 16), _f32),
            pltpu.VMEM((K, 16), _f32),
            pltpu.VMEM((K * 16,), _f32),
            pltpu.VMEM((K, row_w), _f32),
            pltpu.VMEM((K, row_w), _f32),
            pltpu.VMEM((16,), _f32),
            pltpu.VMEM((K * row_w,), _i32),
            pltpu.SemaphoreType.DMA,
            pltpu.SemaphoreType.DMA,
            pltpu.SemaphoreType.DMA,
            pltpu.SemaphoreType.DMA,
            pltpu.SemaphoreType.DMA,
            pltpu.SemaphoreType.DMA,
        ],
    )
    def edge_pass(src_hbm, dst_hbm, tab_hbm, adt_hbm, bound_hbm,
                  tall_hbm,
                  out_hbm,
                  *scratch):
        if stage_tables:
            (tab_src, adt_src, acc_sh, sidx_all, didx_all,
             srows_0, srows_1, srows_2, srows_3,
             adrows_0, adrows_1, adrows_2, adrows_3,
             exbuf, staging_a, staging_b, boundv, tall,
             sem_0, sem_1, sem_2, sem_3, sem_sa, sem_sb) = scratch
        else:
            (acc_sh, sidx_all, didx_all,
             srows_0, srows_1, srows_2, srows_3,
             adrows_0, adrows_1, adrows_2, adrows_3,
             exbuf, staging_a, staging_b, boundv, tall,
             sem_0, sem_1, sem_2, sem_3, sem_sa, sem_sb) = scratch
            tab_src, adt_src = tab_hbm, adt_hbm
        cid = lax.axis_index("c")
        sid = lax.axis_index("s")
        wid = sid * NC + cid
        row0 = sid * RPT
        ebase = wid * EPW

        pltpu.sync_copy(bound_hbm, boundv)
        pltpu.sync_copy(tall_hbm, tall)
        pltpu.sync_copy(src_hbm.at[pl.ds(ebase, EPW)], sidx_all)
        pltpu.sync_copy(dst_hbm.at[pl.ds(ebase, EPW)], didx_all)
        if stage_tables:
            pltpu.sync_copy(tab_hbm.at[pl.ds(row0, RPT)],
                            tab_src.at[pl.ds(row0, RPT)])
            pltpu.sync_copy(adt_hbm.at[pl.ds(row0, RPT)],
                            adt_src.at[pl.ds(row0, RPT)])

        zero16 = jnp.zeros((16,), _f32)

        def zrow(r, carry):
            for j in range(W16):
                staging_a[r, pl.ds(j * 16, 16)] = zero16
            return carry

        lax.fori_loop(0, K, zrow, 0)
        nfull = RPT // K
        rem = RPT - nfull * K
        for r in range(nfull):
            pltpu.sync_copy(staging_a, acc_sh.at[pl.ds(row0 + r * K, K)])
        if rem:
            pltpu.sync_copy(staging_a.at[pl.ds(0, rem)],
                            acc_sh.at[pl.ds(row0 + nfull * K, rem)])
        plsc.subcore_barrier()

        bvec = boundv[...]
        lane = lax.broadcasted_iota(_i32, (16,), 0)

        def fire(c, srows, adrows, sem):
            sl = pl.ds(c * K, K)
            pltpu.async_copy(tab_src.at[sidx_all.at[sl]], srows, sem)
            pltpu.async_copy(adt_src.at[didx_all.at[sl]], adrows, sem)

        def drain(srows, adrows, sem):
            pltpu.make_async_copy(tab_hbm.at[pl.ds(0, K)], srows, sem).wait()
            pltpu.make_async_copy(adt_hbm.at[pl.ds(0, K)], adrows, sem).wait()

        def scat(c, staging, sem_s):
            return pltpu.make_async_copy(
                staging, acc_sh.at[didx_all.at[pl.ds(c * K, K)]], sem_s)

        def compute_scatter(c, srows, adrows, staging, sem_s, guard):
            @pl.when(guard)
            def _():
                scat(c, staging, sem_s).wait()

            @plsc.parallel_loop(0, K, unroll=4)
            def exphase(e):
                asv = srows[e, pl.ds(AS_W0, 16)]
                adv = adrows[e, pl.ds(0, 16)]
                ev = asv + adv
                ev = jnp.where(ev > 0, ev, 0.2 * ev)
                exbuf[pl.ds(e * 16, 16)] = jnp.exp(ev - bvec)

            @plsc.parallel_loop(0, K, unroll=4)
            def msg(e):
                for j in range(W16):
                    lo = j * 16
                    colv = lane + lo
                    idxv = tall[pl.ds(e * row_w + lo, 16)]
                    exg = plsc.load_gather(exbuf, [idxv])
                    if lo + 16 <= H * C:
                        val = srows[e, pl.ds(lo, 16)] * exg
                    elif lo >= H * C:
                        val = jnp.where(colv < H * C + H, exg, 0.0)
                    else:
                        hv = srows[e, pl.ds(lo, 16)]
                        val = jnp.where(colv < H * C, hv * exg,
                                        jnp.where(colv < H * C + H, exg, 0.0))
                    staging[e, pl.ds(lo, 16)] = val

            pltpu.async_copy(staging,
                             acc_sh.at[didx_all.at[pl.ds(c * K, K)]],
                             sem_s, add=True)

        bufs = [(srows_0, adrows_0, sem_0), (srows_1, adrows_1, sem_1),
                (srows_2, adrows_2, sem_2), (srows_3, adrows_3, sem_3)]
        stgs = [(staging_a, sem_sa), (staging_b, sem_sb)]
        DEPTH = 4
        for b in range(DEPTH):
            fire(b, *bufs[b])

        def quad(i, carry):
            for b in range(DEPTH):
                c = DEPTH * i + b
                drain(*bufs[b])
                stg, ssem = stgs[b % 2]
                guard = (i > 0) if b < 2 else jnp.bool_(True)
                compute_scatter(c, bufs[b][0], bufs[b][1], stg, ssem, guard)

                @pl.when(c + DEPTH < NCHUNK)
                def _(sb=bufs[b], cc=c):
                    fire(cc + DEPTH, *sb)
            return carry

        nquad = NCHUNK // DEPTH
        lax.fori_loop(0, nquad, quad, 0)
        for b in range(NCHUNK - DEPTH * nquad):
            c = DEPTH * nquad + b
            drain(*bufs[b])
            stg, ssem = stgs[b % 2]
            compute_scatter(c, bufs[b][0], bufs[b][1], stg, ssem,
                            jnp.bool_(True))
        scat(0, staging_a, sem_sa).wait()
        scat(0, staging_b, sem_sb).wait()

        plsc.subcore_barrier()
        pltpu.sync_copy(acc_sh.at[pl.ds(row0, RPT)],
                        out_hbm.at[cid, pl.ds(row0, RPT)])

    return edge_pass


def _make_edge_pass_1head(row_w, C):
    assert NCHUNK % 2 == 1
    mesh = plsc.VectorSubcoreMesh(core_axis_name="c", subcore_axis_name="s")

    @functools.partial(
        pl.kernel,
        mesh=mesh,
        out_type=jax.ShapeDtypeStruct((NC, N, row_w), _f32),
        compiler_params=pltpu.CompilerParams(
            use_tc_tiling_on_sc=False, needs_layout_passes=False),
        scratch_types=[
            pltpu.VMEM_SHARED((N, row_w), _f32),
            pltpu.VMEM((EPW,), _i32),
            pltpu.VMEM((EPW,), _i32),
            pltpu.VMEM((K, row_w), _f32),
            pltpu.VMEM((K, row_w), _f32),
            pltpu.VMEM((K, row_w), _f32),
            pltpu.VMEM((K, row_w), _f32),
            pltpu.VMEM((N,), _f32),
            pltpu.VMEM((N,), _f32),
            pltpu.VMEM((K,), _f32),
            pltpu.VMEM((K, row_w), _f32),
            pltpu.VMEM((K, row_w), _f32),
            pltpu.VMEM((16,), _f32),
            pltpu.SemaphoreType.DMA,
            pltpu.SemaphoreType.DMA,
            pltpu.SemaphoreType.DMA,
            pltpu.SemaphoreType.DMA,
            pltpu.SemaphoreType.DMA,
            pltpu.SemaphoreType.DMA,
        ],
    )
    def edge_pass(src_hbm, dst_hbm, tab_hbm, asf_hbm, adf_hbm, bound_hbm,
                  out_hbm,
                  acc_sh, sidx_all, didx_all,
                  srows_0, srows_1, srows_2, srows_3,
                  asf, adf, exbuf, staging_a, staging_b, boundv,
                  sem_0, sem_1, sem_2, sem_3, sem_sa, sem_sb):
        cid = lax.axis_index("c")
        sid = lax.axis_index("s")
        wid = sid * NC + cid
        row0 = sid * RPT
        ebase = wid * EPW

        pltpu.sync_copy(bound_hbm, boundv)
        pltpu.sync_copy(src_hbm.at[pl.ds(ebase, EPW)], sidx_all)
        pltpu.sync_copy(dst_hbm.at[pl.ds(ebase, EPW)], didx_all)
        pltpu.sync_copy(asf_hbm, asf)
        pltpu.sync_copy(adf_hbm, adf)

        zero16 = jnp.zeros((16,), _f32)

        def zrow(r, carry):
            staging_a[r, pl.ds(0, 16)] = zero16
            return carry

        lax.fori_loop(0, K, zrow, 0)
        nfull = RPT // K
        rem = RPT - nfull * K
        for r in range(nfull):
            pltpu.sync_copy(staging_a, acc_sh.at[pl.ds(row0 + r * K, K)])
        if rem:
            pltpu.sync_copy(staging_a.at[pl.ds(0, rem)],
                            acc_sh.at[pl.ds(row0 + nfull * K, rem)])
        plsc.subcore_barrier()

        bvec = boundv[...]
        lane = lax.broadcasted_iota(_i32, (16,), 0)

        def fire(c, srows, sem):
            pltpu.async_copy(tab_hbm.at[sidx_all.at[pl.ds(c * K, K)]],
                             srows, sem)

        def drain(srows, sem):
            pltpu.make_async_copy(tab_hbm.at[pl.ds(0, K)], srows, sem).wait()

        def scat(c, staging, sem_s):
            return pltpu.make_async_copy(
                staging, acc_sh.at[didx_all.at[pl.ds(c * K, K)]], sem_s)

        def compute_scatter(c, srows, staging, sem_s, guard):
            @pl.when(guard)
            def _():
                scat(c, staging, sem_s).wait()

            @plsc.parallel_loop(0, K // 16, unroll=2)
            def exphase(g):
                sl16 = pl.ds(c * K + g * 16, 16)
                sv = plsc.load_gather(asf, [sidx_all[sl16]])
                dv = plsc.load_gather(adf, [didx_all[sl16]])
                ev = sv + dv
                ev = jnp.where(ev > 0, ev, 0.2 * ev)
                exbuf[pl.ds(g * 16, 16)] = jnp.exp(ev - bvec)

            @plsc.parallel_loop(0, K, unroll=4)
            def msg(e):
                exg = plsc.load_gather(exbuf, [jnp.broadcast_to(e, (16,))])
                hv = srows[e, pl.ds(0, 16)]
                val = jnp.where(lane < C, hv * exg,
                                jnp.where(lane < C + 1, exg, 0.0))
                staging[e, pl.ds(0, 16)] = val

            pltpu.async_copy(staging,
                             acc_sh.at[didx_all.at[pl.ds(c * K, K)]],
                             sem_s, add=True)

        bufs = [(srows_0, sem_0), (srows_1, sem_1),
                (srows_2, sem_2), (srows_3, sem_3)]
        stgs = [(staging_a, sem_sa), (staging_b, sem_sb)]
        DEPTH = 4
        for b in range(DEPTH):
            fire(b, *bufs[b])

        def quad(i, carry):
            for b in range(DEPTH):
                c = DEPTH * i + b
                drain(*bufs[b])
                stg, ssem = stgs[b % 2]
                guard = (i > 0) if b < 2 else jnp.bool_(True)
                compute_scatter(c, bufs[b][0], stg, ssem, guard)

                @pl.when(c + DEPTH < NCHUNK)
                def _(sb=bufs[b], cc=c):
                    fire(cc + DEPTH, *sb)
            return carry

        nquad = NCHUNK // DEPTH
        lax.fori_loop(0, nquad, quad, 0)
        for b in range(NCHUNK - DEPTH * nquad):
            c = DEPTH * nquad + b
            drain(*bufs[b])
            stg, ssem = stgs[b % 2]
            compute_scatter(c, bufs[b][0], stg, ssem, jnp.bool_(True))
        scat(0, staging_a, sem_sa).wait()
        scat(0, staging_b, sem_sb).wait()

        plsc.subcore_barrier()
        pltpu.sync_copy(acc_sh.at[pl.ds(row0, RPT)],
                        out_hbm.at[cid, pl.ds(row0, RPT)])

    return edge_pass


_edge_pass1 = _make_edge_pass(80, H1, C1, stage_tables=False)
_edge_pass2 = _make_edge_pass_1head(16, NCLASS)
_, _, _TALL1 = _edge_tables(80, H1, C1)
_, _LANE0_2, _TALL2 = _edge_tables(16, 1, NCLASS)


BK = 2000
NBLK = N // BK


def _tcA_body(z_ref, w_ref, as_ref, ad_ref, tab_ref, adt_ref, ms_ref, md_ref):
    i = pl.program_id(0)
    h = jnp.dot(z_ref[...], w_ref[...], preferred_element_type=_f32)
    rows = lax.broadcasted_iota(_i32, (H1 * C1, H1), 0)
    cols = lax.broadcasted_iota(_i32, (H1 * C1, H1), 1)
    G = jnp.where(rows // C1 == cols, 1.0, 0.0).astype(_f32)
    asn = jnp.dot(h * as_ref[...], G, preferred_element_type=_f32)
    adn = jnp.dot(h * ad_ref[...], G, preferred_element_type=_f32)
    tab_ref[...] = jnp.concatenate(
        [h, asn, jnp.zeros((BK, 8), _f32)], axis=1)
    adt_ref[...] = jnp.concatenate([adn, jnp.zeros((BK, 8), _f32)], axis=1)
    mva = jnp.max(asn, axis=0, keepdims=True)
    mvd = jnp.max(adn, axis=0, keepdims=True)

    @pl.when(i == 0)
    def _():
        ms_ref[...] = mva
        md_ref[...] = mvd

    @pl.when(i > 0)
    def _():
        ms_ref[...] = jnp.maximum(ms_ref[...], mva)
        md_ref[...] = jnp.maximum(md_ref[...], mvd)


def _tcA(z, w1, a_s_flat, a_d_flat):
    return pl.pallas_call(
        _tcA_body,
        grid=(NBLK,),
        in_specs=[
            pl.BlockSpec((BK, IN_DIM), lambda i: (i, 0)),
            pl.BlockSpec((IN_DIM, H1 * C1), lambda i: (0, 0)),
            pl.BlockSpec((1, H1 * C1), lambda i: (0, 0)),
            pl.BlockSpec((1, H1 * C1), lambda i: (0, 0)),
        ],
        out_specs=[
            pl.BlockSpec((BK, 80), lambda i: (i, 0)),
            pl.BlockSpec((BK, 16), lambda i: (i, 0)),
            pl.BlockSpec((1, H1), lambda i: (0, 0)),
            pl.BlockSpec((1, H1), lambda i: (0, 0)),
        ],
        out_shape=[
            jax.ShapeDtypeStruct((N, 80), _f32),
            jax.ShapeDtypeStruct((N, 16), _f32),
            jax.ShapeDtypeStruct((1, H1), _f32),
            jax.ShapeDtypeStruct((1, H1), _f32),
        ],
    )(z, w1, a_s_flat, a_d_flat)


def _tcB_body(acc_ref, b1_ref, w2_ref, as2_ref, ad2_ref,
              tab2_ref, ad2t_ref, ms_ref, md_ref):
    i = pl.program_id(0)
    s = acc_ref[0] + acc_ref[1]
    num = s[:, :H1 * C1]
    den = s[:, H1 * C1:H1 * C1 + H1]
    rows = lax.broadcasted_iota(_i32, (H1, H1 * C1), 0)
    cols = lax.broadcasted_iota(_i32, (H1, H1 * C1), 1)
    P = jnp.where(rows == cols // C1, 1.0, 0.0).astype(_f32)
    denb = jnp.dot(den, P, preferred_element_type=_f32)
    x = num / (denb + 1e-16) + b1_ref[...]
    x = jnp.where(x > 0, x, jnp.exp(x) - 1.0)
    h2 = jnp.dot(x, w2_ref[...], preferred_element_type=_f32)
    as2 = jnp.sum(h2 * as2_ref[...], axis=1, keepdims=True)
    ad2 = jnp.sum(h2 * ad2_ref[...], axis=1, keepdims=True)
    tab2_ref[...] = jnp.concatenate(
        [h2, as2, jnp.zeros((BK, 5), _f32)], axis=1)
    ad2t_ref[...] = jnp.concatenate(
        [jnp.zeros((BK, 10), _f32), ad2, jnp.zeros((BK, 5), _f32)], axis=1)
    mva = jnp.max(as2).reshape(1, 1)
    mvd = jnp.max(ad2).reshape(1, 1)

    @pl.when(i == 0)
    def _():
        ms_ref[...] = mva
        md_ref[...] = mvd

    @pl.when(i > 0)
    def _():
        ms_ref[...] = jnp.maximum(ms_ref[...], mva)
        md_ref[...] = jnp.maximum(md_ref[...], mvd)


def _tcB(acc1, b1_row, w2, a_src2, a_dst2):
    return pl.pallas_call(
        _tcB_body,
        grid=(NBLK,),
        in_specs=[
            pl.BlockSpec((NC, BK, 80), lambda i: (0, i, 0)),
            pl.BlockSpec((1, H1 * C1), lambda i: (0, 0)),
            pl.BlockSpec((H1 * C1, NCLASS), lambda i: (0, 0)),
            pl.BlockSpec((1, NCLASS), lambda i: (0, 0)),
            pl.BlockSpec((1, NCLASS), lambda i: (0, 0)),
        ],
        out_specs=[
            pl.BlockSpec((BK, 16), lambda i: (i, 0)),
            pl.BlockSpec((BK, 16), lambda i: (i, 0)),
            pl.BlockSpec((1, 1), lambda i: (0, 0)),
            pl.BlockSpec((1, 1), lambda i: (0, 0)),
        ],
        out_shape=[
            jax.ShapeDtypeStruct((N, 16), _f32),
            jax.ShapeDtypeStruct((N, 16), _f32),
            jax.ShapeDtypeStruct((1, 1), _f32),
            jax.ShapeDtypeStruct((1, 1), _f32),
        ],
    )(acc1, b1_row, w2, a_src2, a_dst2)


def _tcC_body(acc_ref, b2_ref, out_ref):
    s = acc_ref[0] + acc_ref[1]
    lanes = lax.broadcasted_iota(_i32, (BK, 16), 1)
    den = jnp.sum(jnp.where(lanes == NCLASS, s, 0.0), axis=1, keepdims=True)
    o = s / (den + 1e-16) + b2_ref[...]
    om = jnp.where(lanes < NCLASS, o, -1e30)
    m = jnp.max(om, axis=1, keepdims=True)
    ssum = jnp.sum(jnp.where(lanes < NCLASS, jnp.exp(o - m), 0.0),
                   axis=1, keepdims=True)
    out_ref[...] = o - (m + jnp.log(ssum))


def _tcC(acc2, b2_row):
    return pl.pallas_call(
        _tcC_body,
        grid=(NBLK,),
        in_specs=[
            pl.BlockSpec((NC, BK, 16), lambda i: (0, i, 0)),
            pl.BlockSpec((1, 16), lambda i: (0, 0)),
        ],
        out_specs=pl.BlockSpec((BK, 16), lambda i: (i, 0)),
        out_shape=jax.ShapeDtypeStruct((N, 16), _f32),
    )(acc2, b2_row)


def kernel(z, edge_index, W1, a_src1, a_dst1, b1, W2, a_src2, a_dst2, b2):
    src = edge_index[0]
    dst = edge_index[1]

    tab1, ad1t, ms1, md1 = _tcA(z, W1, a_src1.reshape(1, H1 * C1),
                                a_dst1.reshape(1, H1 * C1))
    m1 = ms1 + md1
    bound1 = jnp.where(m1 > 0, m1, 0.2 * m1).reshape(H1)
    b1_16 = jnp.tile(bound1, 2)
    acc1 = _edge_pass1(src, dst, tab1, ad1t, b1_16, _TALL1)

    tab2, ad2t, ms2, md2 = _tcB(acc1, b1.reshape(1, H1 * C1),
                                W2, a_src2, a_dst2)
    m2 = ms2 + md2
    bound2 = jnp.where(m2 > 0, m2, 0.2 * m2)
    b2_16 = jnp.broadcast_to(bound2.reshape(1), (16,))
    acc2 = _edge_pass2(src, dst, tab2, tab2[:, 10], ad2t[:, 10], b2_16)

    b2_row = jnp.concatenate([b2, jnp.zeros((6,), _f32)]).reshape(1, 16)
    outc = _tcC(acc2, b2_row)
    return outc[:, :NCLASS]
